# baseline (speedup 1.0000x reference)
import jax
import jax.numpy as jnp
from jax.experimental import pallas as pl
from jax.experimental.pallas import tpu as pltpu

NUM_TOKENS = 32768
HIDDEN = 2048
NUM_EXPERTS = 64
TOP_K = 8
BLOCK = 512
N_BLOCKS = NUM_TOKENS // BLOCK
NBUF = 6
LOOK = 3


def _in_copy(h_ref, buf, sem_in, c):
    return pltpu.make_async_copy(
        h_ref.at[pl.ds(c * BLOCK, BLOCK), :], buf.at[c % NBUF], sem_in.at[c % NBUF])


def _out_copy(h_out_ref, buf, sem_out, c):
    return pltpu.make_async_copy(
        buf.at[c % NBUF], h_out_ref.at[pl.ds(c * BLOCK, BLOCK), :], sem_out.at[c % NBUF])


def _fused_kernel(h_ref, logits_ref, h_out_ref, w_ref, id_ref,
                  buf, sem_in, sem_out):
    i = pl.program_id(0)

    @pl.when(i == 0)
    def _prime():
        for c in range(LOOK):
            _in_copy(h_ref, buf, sem_in, c).start()

    @pl.when(jnp.logical_and(i + LOOK < N_BLOCKS, i >= LOOK))
    def _recycle_wait():
        _out_copy(h_out_ref, buf, sem_out, i - LOOK).wait()

    @pl.when(i + LOOK < N_BLOCKS)
    def _next_in():
        _in_copy(h_ref, buf, sem_in, i + LOOK).start()

    _in_copy(h_ref, buf, sem_in, i).wait()
    _out_copy(h_out_ref, buf, sem_out, i).start()

    x = logits_ref[pl.ds(i * BLOCK, BLOCK), :]
    xt = x.T
    b = xt.shape[1]
    mx = jnp.max(xt, axis=0, keepdims=True)
    e = jnp.exp(xt - mx)
    probs = e / jnp.sum(e, axis=0, keepdims=True)

    row8 = jax.lax.broadcasted_iota(jnp.int32, (TOP_K, b), 0)
    row64 = jax.lax.broadcasted_iota(jnp.int32, (NUM_EXPERTS, b), 0)
    vals = jnp.zeros((TOP_K, b), dtype=jnp.float32)
    ids = jnp.zeros((TOP_K, b), dtype=jnp.int32)
    cur = probs
    for j in range(TOP_K):
        m = jnp.max(cur, axis=0, keepdims=True)
        a = jnp.argmax(cur, axis=0).astype(jnp.int32)
        a2 = a[None, :]
        vals = jnp.where(row8 == j, m, vals)
        ids = jnp.where(row8 == j, a2, ids)
        cur = jnp.where(row64 == a2, -1.0, cur)
    w = vals / jnp.sum(vals, axis=0, keepdims=True)
    w_ref[:, pl.ds(i * BLOCK, BLOCK)] = w
    id_ref[:, pl.ds(i * BLOCK, BLOCK)] = ids

    @pl.when(i == N_BLOCKS - 1)
    def _drain():
        for c in range(N_BLOCKS - 2 * LOOK, N_BLOCKS):
            _out_copy(h_out_ref, buf, sem_out, c).wait()


def kernel(hidden_states, router_logits):
    grid = (N_BLOCKS,)
    h_out, w_t, ids_t = pl.pallas_call(
        _fused_kernel,
        grid=grid,
        in_specs=[
            pl.BlockSpec(memory_space=pl.ANY),
            pl.BlockSpec((NUM_TOKENS, NUM_EXPERTS), lambda i: (0, 0)),
        ],
        out_specs=[
            pl.BlockSpec(memory_space=pl.ANY),
            pl.BlockSpec((TOP_K, NUM_TOKENS), lambda i: (0, 0)),
            pl.BlockSpec((TOP_K, NUM_TOKENS), lambda i: (0, 0)),
        ],
        out_shape=[
            jax.ShapeDtypeStruct((NUM_TOKENS, HIDDEN), jnp.float32),
            jax.ShapeDtypeStruct((TOP_K, NUM_TOKENS), jnp.float32),
            jax.ShapeDtypeStruct((TOP_K, NUM_TOKENS), jnp.int32),
        ],
        scratch_shapes=[
            pltpu.VMEM((NBUF, BLOCK, HIDDEN), jnp.float32),
            pltpu.SemaphoreType.DMA((NBUF,)),
            pltpu.SemaphoreType.DMA((NBUF,)),
        ],
    )(hidden_states, router_logits)
    return h_out, w_t.T, ids_t.T


# speedup vs baseline: 1.0984x; 1.0984x over previous
import jax
import jax.numpy as jnp
from jax.experimental import pallas as pl

NUM_TOKENS = 32768
HIDDEN = 2048
NUM_EXPERTS = 64
TOP_K = 8
BLOCK = 1024
N_BLOCKS = NUM_TOKENS // BLOCK


def _fused_kernel(h_ref, logits_ref, h_out_ref, w_ref, id_ref):
    h_out_ref[...] = h_ref[...]

    i = pl.program_id(0)
    xt = logits_ref[:, pl.ds(i * BLOCK, BLOCK)]
    b = xt.shape[1]
    mx = jnp.max(xt, axis=0, keepdims=True)
    e = jnp.exp(xt - mx)
    probs = e / jnp.sum(e, axis=0, keepdims=True)

    row8 = jax.lax.broadcasted_iota(jnp.int32, (TOP_K, b), 0)
    row64 = jax.lax.broadcasted_iota(jnp.int32, (NUM_EXPERTS, b), 0)
    vals = jnp.zeros((TOP_K, b), dtype=jnp.float32)
    ids = jnp.zeros((TOP_K, b), dtype=jnp.int32)
    cur = probs
    for j in range(TOP_K):
        m = jnp.max(cur, axis=0, keepdims=True)
        a = jnp.argmax(cur, axis=0).astype(jnp.int32)
        a2 = a[None, :]
        vals = jnp.where(row8 == j, m, vals)
        ids = jnp.where(row8 == j, a2, ids)
        cur = jnp.where(row64 == a2, -1.0, cur)
    w = vals / jnp.sum(vals, axis=0, keepdims=True)
    w_ref[:, pl.ds(i * BLOCK, BLOCK)] = w
    id_ref[:, pl.ds(i * BLOCK, BLOCK)] = ids


def kernel(hidden_states, router_logits):
    grid = (N_BLOCKS,)
    h_out, w_t, ids_t = pl.pallas_call(
        _fused_kernel,
        grid=grid,
        in_specs=[
            pl.BlockSpec((BLOCK, HIDDEN), lambda i: (i, 0)),
            pl.BlockSpec((NUM_EXPERTS, NUM_TOKENS), lambda i: (0, 0)),
        ],
        out_specs=[
            pl.BlockSpec((BLOCK, HIDDEN), lambda i: (i, 0)),
            pl.BlockSpec((TOP_K, NUM_TOKENS), lambda i: (0, 0)),
            pl.BlockSpec((TOP_K, NUM_TOKENS), lambda i: (0, 0)),
        ],
        out_shape=[
            jax.ShapeDtypeStruct((NUM_TOKENS, HIDDEN), jnp.float32),
            jax.ShapeDtypeStruct((TOP_K, NUM_TOKENS), jnp.float32),
            jax.ShapeDtypeStruct((TOP_K, NUM_TOKENS), jnp.int32),
        ],
    )(hidden_states, router_logits.T)
    return h_out, w_t.T, ids_t.T
